# baseline (speedup 1.0000x reference)
import functools

import jax
import jax.numpy as jnp
import numpy as np
from jax.experimental import pallas as pl
from jax.experimental.pallas import tpu as pltpu

_BLK = 2048


@functools.cache
def _noise_sample(n, e):
    with jax.ensure_compile_time_eval():
        return np.asarray(
            jax.random.normal(jax.random.key(42), (n, e), dtype=jnp.float32))


def _gating_kernel(x_ref, wg_ref, bg_ref, wn_ref, bn_ref, noise_ref,
                   weights_ref, logits_ref):
    x = x_ref[...]
    dn = (((1,), (1,)), ((), ()))
    clean = jax.lax.dot_general(
        x, wg_ref[...], dimension_numbers=dn,
        preferred_element_type=jnp.float32) + bg_ref[...]
    raw_noise = jax.lax.dot_general(
        x, wn_ref[...], dimension_numbers=dn,
        preferred_element_type=jnp.float32) + bn_ref[...]
    noise_std = jnp.logaddexp(raw_noise, 0.0)
    logits = clean + noise_ref[...] * noise_std
    logits_ref[...] = logits
    m = jnp.max(logits, axis=-1, keepdims=True)
    ex = jnp.exp(logits - m)
    weights_ref[...] = ex / jnp.sum(ex, axis=-1, keepdims=True)


def kernel(x, Wg, bg, Wn, bn):
    n, d = x.shape
    e = Wg.shape[0]
    noise = jnp.asarray(_noise_sample(n, e))
    grid = (n // _BLK,)
    out_shape = [
        jax.ShapeDtypeStruct((n, e), jnp.float32),
        jax.ShapeDtypeStruct((n, e), jnp.float32),
    ]
    weights, logits = pl.pallas_call(
        _gating_kernel,
        grid=grid,
        in_specs=[
            pl.BlockSpec((_BLK, d), lambda i: (i, 0)),
            pl.BlockSpec((e, d), lambda i: (0, 0)),
            pl.BlockSpec((1, e), lambda i: (0, 0)),
            pl.BlockSpec((e, d), lambda i: (0, 0)),
            pl.BlockSpec((1, e), lambda i: (0, 0)),
            pl.BlockSpec((_BLK, e), lambda i: (i, 0)),
        ],
        out_specs=[
            pl.BlockSpec((_BLK, e), lambda i: (i, 0)),
            pl.BlockSpec((_BLK, e), lambda i: (i, 0)),
        ],
        out_shape=out_shape,
        compiler_params=pltpu.CompilerParams(
            dimension_semantics=("parallel",),
        ),
    )(x, Wg, bg.reshape(1, e), Wn, bn.reshape(1, e), noise)
    return (weights, logits)


# speedup vs baseline: 1.0239x; 1.0239x over previous
import functools

import jax
import jax.numpy as jnp
import numpy as np
from jax.experimental import pallas as pl
from jax.experimental.pallas import tpu as pltpu

_BLK = 1024


@functools.cache
def _noise_sample(n, e):
    with jax.ensure_compile_time_eval():
        return np.asarray(
            jax.random.normal(jax.random.key(42), (n, e), dtype=jnp.float32))


def _gating_kernel(x_ref, wg_ref, bg_ref, wn_ref, bn_ref, noise_ref,
                   weights_ref, logits_ref):
    x = x_ref[...]
    dn = (((1,), (1,)), ((), ()))
    clean = jax.lax.dot_general(
        x, wg_ref[...], dimension_numbers=dn,
        preferred_element_type=jnp.float32) + bg_ref[...]
    raw_noise = jax.lax.dot_general(
        x, wn_ref[...], dimension_numbers=dn,
        preferred_element_type=jnp.float32) + bn_ref[...]
    noise_std = jnp.logaddexp(raw_noise, 0.0)
    logits = clean + noise_ref[...] * noise_std
    logits_ref[...] = logits
    m = jnp.max(logits, axis=-1, keepdims=True)
    ex = jnp.exp(logits - m)
    weights_ref[...] = ex / jnp.sum(ex, axis=-1, keepdims=True)


def kernel(x, Wg, bg, Wn, bn):
    n, d = x.shape
    e = Wg.shape[0]
    noise = jnp.asarray(_noise_sample(n, e))
    grid = (n // _BLK,)
    out_shape = [
        jax.ShapeDtypeStruct((n, e), jnp.float32),
        jax.ShapeDtypeStruct((n, e), jnp.float32),
    ]
    weights, logits = pl.pallas_call(
        _gating_kernel,
        grid=grid,
        in_specs=[
            pl.BlockSpec((_BLK, d), lambda i: (i, 0)),
            pl.BlockSpec((e, d), lambda i: (0, 0)),
            pl.BlockSpec((1, e), lambda i: (0, 0)),
            pl.BlockSpec((e, d), lambda i: (0, 0)),
            pl.BlockSpec((1, e), lambda i: (0, 0)),
            pl.BlockSpec((_BLK, e), lambda i: (i, 0)),
        ],
        out_specs=[
            pl.BlockSpec((_BLK, e), lambda i: (i, 0)),
            pl.BlockSpec((_BLK, e), lambda i: (i, 0)),
        ],
        out_shape=out_shape,
        compiler_params=pltpu.CompilerParams(
            dimension_semantics=("parallel",),
        ),
    )(x, Wg, bg.reshape(1, e), Wn, bn.reshape(1, e), noise)
    return (weights, logits)
